# baseline (speedup 1.0000x reference)
import functools

import jax
import jax.numpy as jnp
from jax import lax
from jax.experimental import pallas as pl
from jax.experimental.pallas import tpu as pltpu, tpu_sc as plsc

N_NODES = 10000
N_EDGES = 320000
D_FEAT = 128

_NC = 2
_NS = 16
_LANES = 16

_K = 80
_TB = 4096
_E_PAD = _TB * _K
_C0 = 216
_C0_HALVES = ((0, 112), (112, 104))
_C1CNT_HALVES = ((0, 128), (128, 128))
_C1SUM = 40
_C1_BASE = _NS * _C0
_N_PAD = 10112
_DUMP_ROW = _N_PAD - 1
_ROWS_PER_TILE = _N_PAD // _NS

_IDXW = 112 * _K
_SEGW = 128

_mesh = plsc.VectorSubcoreMesh(core_axis_name="c", subcore_axis_name="s")


def _fill_2d(ref, nrows, ncols, val):
    v = jnp.full((_LANES,), val, jnp.float32)

    def row(i, _):
        for j in range(ncols // _LANES):
            ref[i, pl.ds(j * _LANES, _LANES)] = v
        return 0

    lax.fori_loop(0, nrows, row, 0)


@functools.partial(
    pl.kernel,
    out_type=(
        jax.ShapeDtypeStruct((_N_PAD, D_FEAT), jnp.float32),
        jax.ShapeDtypeStruct((_N_PAD, D_FEAT), jnp.float32),
        jax.ShapeDtypeStruct((_N_PAD, D_FEAT), jnp.float32),
    ),
    mesh=_mesh,
    scratch_types=(
        pltpu.VMEM((_IDXW,), jnp.int32),
        pltpu.VMEM((_SEGW, _K), jnp.int32),
        pltpu.VMEM((_K, D_FEAT), jnp.float32),
        pltpu.VMEM((_K, D_FEAT), jnp.float32),
        pltpu.VMEM_SHARED((_N_PAD, D_FEAT), jnp.float32),
        pltpu.SemaphoreType.DMA,
        pltpu.SemaphoreType.DMA,
        pltpu.SemaphoreType.DMA,
        pltpu.SemaphoreType.DMA,
    ),
)
def _phase1(feat_hbm, nidx_hbm, seg_hbm, sums0_out, sums1_out, cnts_out,
            idx_v, seg_v, rows0, rows1, acc, gs0, gs1, ss0, ss1):
    cid = lax.axis_index("c")
    sid = lax.axis_index("s")
    r0 = sid * _ROWS_PER_TILE
    nzb = _ROWS_PER_TILE // _K
    nzt = _ROWS_PER_TILE - nzb * _K
    rows = (rows0, rows1)
    gsem = (gs0, gs1)
    ssem = (ss0, ss1)

    def zero_acc():
        for i in range(nzb):
            pltpu.sync_copy(rows0, acc.at[pl.ds(r0 + i * _K, _K)])
        pltpu.sync_copy(rows0.at[pl.ds(0, nzt)],
                        acc.at[pl.ds(r0 + nzb * _K, nzt)])

    def gather_src(g):
        return feat_hbm.at[idx_v.at[pl.ds(g * _K, _K)]]

    def start_gather(g, b):
        pltpu.async_copy(gather_src(g), rows[b], gsem[b])

    def wait_gather(g, b):
        pltpu.make_async_copy(gather_src(g), rows[b], gsem[b]).wait()

    def start_scatter(g, b):
        pltpu.async_copy(rows[b], acc.at[seg_v.at[g]], ssem[b], add=True)

    def wait_scatter(b):
        pltpu.make_async_copy(rows[b], acc.at[seg_v.at[0]], ssem[b]).wait()

    def start_scatter_ones(g, b):
        pltpu.async_copy(rows0, acc.at[seg_v.at[g]], ssem[b], add=True)

    def wait_scatter_ones(b):
        pltpu.make_async_copy(rows0, acc.at[seg_v.at[0]], ssem[b]).wait()

    def load_tables(a, nb, with_idx):
        if with_idx:
            pltpu.sync_copy(nidx_hbm.at[pl.ds(a * _K, nb * _K)],
                            idx_v.at[pl.ds(0, nb * _K)])
        pltpu.sync_copy(seg_hbm.at[pl.ds(a, nb)], seg_v.at[pl.ds(0, nb)])

    def run_sums(nb):
        start_gather(0, 0)

        def pair(i, _):
            for b in range(2):
                g = 2 * i + b
                b1 = (b + 1) % 2
                pred = jnp.logical_and(g >= 1, g + 1 < nb)

                @pl.when(pred)
                def _():
                    wait_scatter(b1)

                @pl.when(g + 1 < nb)
                def _():
                    start_gather(g + 1, b1)

                wait_gather(g, b)
                start_scatter(g, b)
            return 0

        lax.fori_loop(0, nb // 2, pair, 0)
        wait_scatter(0)
        wait_scatter(1)

    def run_counts(nb):
        def pair(i, _):
            for b in range(2):
                g = 2 * i + b

                @pl.when(g >= 2)
                def _():
                    wait_scatter_ones(b)

                start_scatter_ones(g, b)
            return 0

        lax.fori_loop(0, nb // 2, pair, 0)
        wait_scatter_ones(0)
        wait_scatter_ones(1)

    @pl.when(cid == 0)
    def _():
        _fill_2d(rows0, _K, D_FEAT, 0.0)
        zero_acc()
        plsc.subcore_barrier()

        for off, nb in _C0_HALVES:
            load_tables(sid * _C0 + off, nb, True)
            run_sums(nb)

        plsc.subcore_barrier()
        pltpu.sync_copy(acc.at[pl.ds(r0, _ROWS_PER_TILE)],
                        sums0_out.at[pl.ds(r0, _ROWS_PER_TILE)])

    @pl.when(cid == 1)
    def _():
        _fill_2d(rows0, _K, D_FEAT, 0.0)
        zero_acc()
        _fill_2d(rows0, _K, D_FEAT, 1.0)
        plsc.subcore_barrier()

        for off, nb in _C1CNT_HALVES:
            load_tables(sid * 2 * _SEGW + off, nb, False)
            run_counts(nb)

        plsc.subcore_barrier()
        pltpu.sync_copy(acc.at[pl.ds(r0, _ROWS_PER_TILE)],
                        cnts_out.at[pl.ds(r0, _ROWS_PER_TILE)])
        plsc.subcore_barrier()

        _fill_2d(rows0, _K, D_FEAT, 0.0)
        zero_acc()
        plsc.subcore_barrier()

        load_tables(_C1_BASE + sid * _C1SUM, _C1SUM, True)
        run_sums(_C1SUM)

        plsc.subcore_barrier()
        pltpu.sync_copy(acc.at[pl.ds(r0, _ROWS_PER_TILE)],
                        sums1_out.at[pl.ds(r0, _ROWS_PER_TILE)])


_BLK = 2000


def _combine_body(s0_ref, s1_ref, cnts_ref, out_ref):
    s = s0_ref[...] + s1_ref[...]
    c = cnts_ref[...]
    out_ref[...] = jnp.where(c > 0.0, s / jnp.maximum(c, 1.0), 0.0)


_combine = pl.pallas_call(
    _combine_body,
    grid=(N_NODES // _BLK,),
    in_specs=[
        pl.BlockSpec((_BLK, D_FEAT), lambda i: (i, 0)),
        pl.BlockSpec((_BLK, D_FEAT), lambda i: (i, 0)),
        pl.BlockSpec((_BLK, 1), lambda i: (i, 0)),
    ],
    out_specs=pl.BlockSpec((_BLK, D_FEAT), lambda i: (i, 0)),
    out_shape=jax.ShapeDtypeStruct((N_NODES, D_FEAT), jnp.float32),
)


def kernel(features, neighbor_idx, segment_ids, num_samples):
    del num_samples
    pad = _E_PAD - N_EDGES
    nidx = jnp.pad(neighbor_idx, (0, pad), constant_values=0)
    seg = jnp.pad(segment_ids, (0, pad),
                  constant_values=_DUMP_ROW).reshape(_TB, _K)
    sums0, sums1, cnts = _phase1(features, nidx, seg)
    cnts_col = cnts[:N_NODES, 0:1]
    return _combine(sums0, sums1, cnts_col)


# speedup vs baseline: 2.3045x; 2.3045x over previous
import functools

import jax
import jax.numpy as jnp
from jax import lax
from jax.experimental import pallas as pl
from jax.experimental.pallas import tpu as pltpu, tpu_sc as plsc

N_NODES = 10000
N_EDGES = 320000
D_FEAT = 128

_NC = 2
_NS = 16
_LANES = 16

_EPT = N_EDGES // _NS
_K = 80
_NBH = 125
_HALF = _NBH * _K
_N_PAD = 10112
_ROWS_PER_TILE = _N_PAD // _NS

_mesh = plsc.VectorSubcoreMesh(core_axis_name="c", subcore_axis_name="s")


def _fill_2d(ref, nrows, ncols, val):
    v = jnp.full((_LANES,), val, jnp.float32)

    def row(i, _):
        for j in range(ncols // _LANES):
            ref[i, pl.ds(j * _LANES, _LANES)] = v
        return 0

    lax.fori_loop(0, nrows, row, 0)


@functools.partial(
    pl.kernel,
    out_type=(
        jax.ShapeDtypeStruct((_N_PAD, D_FEAT), jnp.float32),
        jax.ShapeDtypeStruct((_N_PAD, D_FEAT), jnp.float32),
    ),
    mesh=_mesh,
    scratch_types=(
        pltpu.VMEM((_HALF,), jnp.int32),
        pltpu.VMEM((_NBH, _K), jnp.int32),
        pltpu.VMEM((_K, D_FEAT), jnp.float32),
        pltpu.VMEM((_K, D_FEAT), jnp.float32),
        pltpu.VMEM_SHARED((_N_PAD, D_FEAT), jnp.float32),
        pltpu.SemaphoreType.DMA,
        pltpu.SemaphoreType.DMA,
        pltpu.SemaphoreType.DMA,
        pltpu.SemaphoreType.DMA,
    ),
)
def _phase1(feat_hbm, nidx_hbm, seg_hbm, sums_out, cnts_out,
            idx_v, seg_v, rows0, rows1, acc, gs0, gs1, ss0, ss1):
    cid = lax.axis_index("c")
    sid = lax.axis_index("s")
    r0 = sid * _ROWS_PER_TILE
    nzb = _ROWS_PER_TILE // _K
    nzt = _ROWS_PER_TILE - nzb * _K
    rows = (rows0, rows1)
    gsem = (gs0, gs1)
    ssem = (ss0, ss1)

    def zero_acc():
        for i in range(nzb):
            pltpu.sync_copy(rows0, acc.at[pl.ds(r0 + i * _K, _K)])
        pltpu.sync_copy(rows0.at[pl.ds(0, nzt)],
                        acc.at[pl.ds(r0 + nzb * _K, nzt)])

    def gather_src(g):
        return feat_hbm.at[idx_v.at[pl.ds(g * _K, _K)]]

    def start_gather(g, b):
        pltpu.async_copy(gather_src(g), rows[b], gsem[b])

    def wait_gather(g, b):
        pltpu.make_async_copy(gather_src(g), rows[b], gsem[b]).wait()

    def start_scatter(g, b):
        pltpu.async_copy(rows[b], acc.at[seg_v.at[g]], ssem[b], add=True)

    def wait_scatter(b):
        pltpu.make_async_copy(rows[b], acc.at[seg_v.at[0]], ssem[b]).wait()

    def start_scatter_ones(g, b):
        pltpu.async_copy(rows0, acc.at[seg_v.at[g]], ssem[b], add=True)

    def wait_scatter_ones(b):
        pltpu.make_async_copy(rows0, acc.at[seg_v.at[0]], ssem[b]).wait()

    @pl.when(cid == 0)
    def _():
        _fill_2d(rows0, _K, D_FEAT, 0.0)
        zero_acc()
        plsc.subcore_barrier()

        for half in range(2):
            base = sid * _EPT + half * _HALF
            pltpu.sync_copy(nidx_hbm.at[pl.ds(base, _HALF)], idx_v)
            pltpu.sync_copy(seg_hbm.at[2 * sid + half], seg_v)

            start_gather(0, 0)

            def pair_a(i, _):
                for b in range(2):
                    g = 2 * i + b
                    b1 = (b + 1) % 2
                    pred = jnp.logical_and(g >= 1, g + 1 < _NBH)

                    @pl.when(pred)
                    def _():
                        wait_scatter(b1)

                    @pl.when(g + 1 < _NBH)
                    def _():
                        start_gather(g + 1, b1)

                    wait_gather(g, b)
                    start_scatter(g, b)
                return 0

            lax.fori_loop(0, _NBH // 2, pair_a, 0)
            wait_gather(_NBH - 1, 0)
            start_scatter(_NBH - 1, 0)
            wait_scatter(1)
            wait_scatter(0)

        plsc.subcore_barrier()
        pltpu.sync_copy(acc.at[pl.ds(r0, _ROWS_PER_TILE)],
                        sums_out.at[pl.ds(r0, _ROWS_PER_TILE)])

    @pl.when(cid == 1)
    def _():
        _fill_2d(rows0, _K, D_FEAT, 0.0)
        zero_acc()
        _fill_2d(rows0, _K, D_FEAT, 1.0)
        plsc.subcore_barrier()

        for half in range(2):
            pltpu.sync_copy(seg_hbm.at[2 * sid + half], seg_v)

            def pair_b(i, _):
                for b in range(2):
                    g = 2 * i + b

                    @pl.when(g >= 2)
                    def _():
                        wait_scatter_ones(b)

                    start_scatter_ones(g, b)
                return 0

            lax.fori_loop(0, _NBH // 2, pair_b, 0)
            wait_scatter_ones(0)
            start_scatter_ones(_NBH - 1, 0)
            wait_scatter_ones(1)
            wait_scatter_ones(0)

        plsc.subcore_barrier()
        pltpu.sync_copy(acc.at[pl.ds(r0, _ROWS_PER_TILE)],
                        cnts_out.at[pl.ds(r0, _ROWS_PER_TILE)])


_BLK = 2000


def _combine_body(sums_ref, cnts_ref, out_ref):
    s = sums_ref[...]
    c = cnts_ref[...]
    out_ref[...] = jnp.where(c > 0.0, s / jnp.maximum(c, 1.0), 0.0)


_combine = pl.pallas_call(
    _combine_body,
    grid=(N_NODES // _BLK,),
    in_specs=[
        pl.BlockSpec((_BLK, D_FEAT), lambda i: (i, 0)),
        pl.BlockSpec((_BLK, 1), lambda i: (i, 0)),
    ],
    out_specs=pl.BlockSpec((_BLK, D_FEAT), lambda i: (i, 0)),
    out_shape=jax.ShapeDtypeStruct((N_NODES, D_FEAT), jnp.float32),
)


def kernel(features, neighbor_idx, segment_ids, num_samples):
    del num_samples
    seg = segment_ids.reshape(_NS * 2, _NBH, _K)
    sums, cnts = _phase1(features, neighbor_idx, seg)
    cnts_col = cnts[:N_NODES, 0:1]
    return _combine(sums, cnts_col)
